# baseline (speedup 1.0000x reference)
import jax
import jax.numpy as jnp
from jax import lax
from jax.experimental import pallas as pl
from jax.experimental.pallas import tpu as pltpu
from jax.experimental.pallas import tpu_sc as plsc

_N, _K, _C, _G = 10000, 16, 128, 8

_TW = 128
_OC = 64
_OKW = 72
_NPAD = 10240

_NW = 32
_CHUNK = 40
_EPW = 5120
_EPAD = _NW * _EPW
_AW = 16

_BN = 1000
_BC = 80


def _relu(x):
    return jnp.maximum(x, 0.0)


def _dot(a, b):
    return jnp.dot(a, b, preferred_element_type=jnp.float32)


def _bn_scale():
    return 1.0 / jnp.sqrt(jnp.float32(1.0) + jnp.float32(1e-5))


def _pack_bf16(v):
    bits = lax.bitcast_convert_type(v, jnp.uint32)
    c = v.shape[-1] // 2
    lo = (bits[:, 0:c] + jnp.uint32(0x8000)) >> jnp.uint32(16)
    hi = (bits[:, c:2 * c] + jnp.uint32(0x8000)) & jnp.uint32(0xFFFF0000)
    return lax.bitcast_convert_type(hi | lo, jnp.float32)


def _unpack_bf16(p):
    bits = lax.bitcast_convert_type(p, jnp.uint32)
    lo = lax.bitcast_convert_type(bits << jnp.uint32(16), jnp.float32)
    hi = lax.bitcast_convert_type(bits & jnp.uint32(0xFFFF0000), jnp.float32)
    return jnp.concatenate([lo, hi], axis=-1)


def _pre_body(feat_ref, fc1w_ref, wq_ref, bq_ref, wk_ref, bk_ref, wv_ref,
              bv_ref, ww1_ref, v_ref, kw_ref, qw_ref):
    s0 = _bn_scale()
    x = _relu(s0 * _dot(feat_ref[...], fc1w_ref[...]))
    q = _relu(s0 * (_dot(x, wq_ref[...]) + bq_ref[...]))
    k = _relu(s0 * (_dot(x, wk_ref[...]) + bk_ref[...]))
    v_ref[...] = _pack_bf16(_dot(x, wv_ref[...]) + bv_ref[...])
    kw_ref[...] = _dot(k, ww1_ref[...])
    qw_ref[...] = _dot(q, ww1_ref[...])


def _pre(feat, fc1w, Wq, bq, Wk, bk, Wv, bv, ww1):
    n, c = feat.shape
    g = ww1.shape[1]
    grid = (n // _BN,)
    full = lambda shape: pl.BlockSpec(shape, lambda i: (0, 0))
    blocked = lambda w: pl.BlockSpec((_BN, w), lambda i: (i, 0))
    return pl.pallas_call(
        _pre_body,
        grid=grid,
        in_specs=[blocked(c), full((c, c)), full((c, c)), full((1, c)),
                  full((c, c)), full((1, c)), full((c, c)), full((1, c)),
                  full((c, g))],
        out_specs=[blocked(c // 2), blocked(g), blocked(g)],
        out_shape=[jax.ShapeDtypeStruct((n, c // 2), jnp.float32),
                   jax.ShapeDtypeStruct((n, g), jnp.float32),
                   jax.ShapeDtypeStruct((n, g), jnp.float32)],
        compiler_params=pltpu.CompilerParams(
            dimension_semantics=("parallel",)),
    )(feat, fc1w, Wq, bq.reshape(1, c), Wk, bk.reshape(1, c), Wv,
      bv.reshape(1, c), ww1)


_NSLOT = 8
_LOOK = 6
_FAST_CID = 0
_CF = 128
_CS = 128


def _gather_body(tbl_hbm, idx_hbm, tblg_out, idxall, vbuf, *sems):
    gsem = sems[:_NSLOT]
    wsem = sems[_NSLOT:]
    cid = lax.axis_index("c")
    sid = lax.axis_index("s")
    is_fast = cid == _FAST_CID
    nch = lax.select(is_fast, jnp.int32(_CF), jnp.int32(_CS))
    rowbase = lax.select(is_fast, sid * _CF, 16 * _CF + sid * _CS)
    base = rowbase * _CHUNK

    @pl.when(is_fast)
    def _():
        pltpu.sync_copy(idx_hbm.at[pl.ds(rowbase, _CF)],
                        idxall.at[pl.ds(0, _CF)])

    if _CS > 0:
        @pl.when(jnp.logical_not(is_fast))
        def _():
            pltpu.sync_copy(idx_hbm.at[pl.ds(rowbase, _CS)],
                            idxall.at[pl.ds(0, _CS)])

    def start_g(slot, chunk):
        pltpu.async_copy(tbl_hbm.at[idxall.at[chunk]], vbuf.at[slot],
                         gsem[slot])

    def wait_g(slot, chunk):
        pltpu.make_async_copy(tbl_hbm.at[idxall.at[chunk]], vbuf.at[slot],
                              gsem[slot]).wait()

    def start_w(slot, chunk):
        off = base + chunk * _CHUNK
        pltpu.async_copy(vbuf.at[slot], tblg_out.at[pl.ds(off, _CHUNK)],
                         wsem[slot])

    def wait_w(slot, chunk):
        off = base + chunk * _CHUNK
        pltpu.make_async_copy(vbuf.at[slot], tblg_out.at[pl.ds(off, _CHUNK)],
                              wsem[slot]).wait()

    @pl.when(nch > 0)
    def _():
        for c in range(_LOOK):
            start_g(c % _NSLOT, c)

    def body(grp, carry):
        for j in range(_NSLOT):
            c = grp * _NSLOT + j
            wait_g(j, c)
            start_w(j, c)
            nj = (j + _LOOK) % _NSLOT

            @pl.when(c + _LOOK < nch)
            def _():
                @pl.when(c >= _NSLOT - _LOOK)
                def _():
                    wait_w(nj, c + _LOOK - _NSLOT)
                start_g(nj, c + _LOOK)
        return carry

    lax.fori_loop(0, nch // _NSLOT, body, 0)

    @pl.when(nch > 0)
    def _():
        for i in range(_LOOK):
            wait_w(_NSLOT - _LOOK + i, nch - _LOOK + i)


def _gather(tbl, idx_grp):
    mesh = plsc.VectorSubcoreMesh(core_axis_name="c", subcore_axis_name="s")
    f = pl.kernel(
        _gather_body,
        out_type=jax.ShapeDtypeStruct((_EPAD, _TW), jnp.float32),
        mesh=mesh,
        scratch_types=([pltpu.VMEM((_CF, _CHUNK), jnp.int32),
                        pltpu.VMEM((_NSLOT, _CHUNK, _TW), jnp.float32)]
                       + [pltpu.SemaphoreType.DMA] * (2 * _NSLOT)),
    )
    return f(tbl, idx_grp)


def _post_body(tblg_ref, ctr_ref, feat_ref, pw1_ref, pb1_ref,
               pw2_ref, pb2_ref, ww1_ref, wb1_ref, ww2_ref, wb2_ref,
               fc3w_ref, out_ref):
    s0 = _bn_scale()
    bc, c = out_ref.shape
    k = _K
    g = _G
    e = bc * k

    tblg = tblg_ref[...]
    auxg = tblg[:, _OC:_OC + _AW]
    ctr = ctr_ref[...]
    crep = jnp.broadcast_to(ctr[:, None, :], (bc, k, _AW)).reshape(e, _AW)

    pos8 = auxg[:, 0:8] - crep[:, 0:8]
    h = _relu(s0 * (_dot(pos8, pw1_ref[...]) + pb1_ref[...]))
    peb = _dot(h, pw2_ref[...]) + pb2_ref[...]

    kwg = auxg[:, 8:16]
    qwr = crep[:, 8:16]
    lg = kwg - qwr + _dot(peb, ww1_ref[...]) + wb1_ref[...]
    t = _relu(s0 * lg)
    wl = (_dot(t, ww2_ref[...]) + wb2_ref[...]).reshape(bc, k, g)

    m = jnp.max(wl, axis=1, keepdims=True)
    ex = jnp.exp(wl - m)
    w = (ex / jnp.sum(ex, axis=1, keepdims=True)).reshape(e, g)

    gid = lax.broadcasted_iota(jnp.int32, (g, c), 0)
    chid = lax.broadcasted_iota(jnp.int32, (g, c), 1) // (c // g)
    rexp = (gid == chid).astype(jnp.float32)
    wexp = _dot(w, rexp)

    val = _unpack_bf16(tblg[:, 0:_OC]) + peb
    attn = jnp.sum((val * wexp).reshape(bc, k, c), axis=1)
    ao = _relu(s0 * attn)
    out_ref[...] = _relu(feat_ref[...] + s0 * _dot(ao, fc3w_ref[...]))


def _post(tblg, ctr, feat, pw1p, pb1, pw2, pb2, ww1, wb1, ww2, wb2, fc3w):
    n, c = feat.shape
    g = ww1.shape[1]
    grid = (n // _BC,)
    full = lambda shape: pl.BlockSpec(shape, lambda i: (0, 0))
    return pl.pallas_call(
        _post_body,
        grid=grid,
        in_specs=[pl.BlockSpec((_BC * _K, _TW), lambda i: (i, 0)),
                  pl.BlockSpec((_BC, _AW), lambda i: (i, 0)),
                  pl.BlockSpec((_BC, c), lambda i: (i, 0)),
                  full((8, c)), full((1, c)), full((c, c)), full((1, c)),
                  full((c, g)), full((1, g)), full((g, g)), full((1, g)),
                  full((c, c))],
        out_specs=pl.BlockSpec((_BC, c), lambda i: (i, 0)),
        out_shape=jax.ShapeDtypeStruct((n, c), jnp.float32),
        compiler_params=pltpu.CompilerParams(
            dimension_semantics=("parallel",)),
    )(tblg, ctr, feat, pw1p, pb1.reshape(1, c), pw2, pb2.reshape(1, c),
      ww1, wb1.reshape(1, g), ww2, wb2.reshape(1, g), fc3w)


def kernel(feat, coord, reference_index, Wq, bq, Wk, bk, Wv, bv, pw1, pb1,
           pw2, pb2, ww1, wb1, ww2, wb2, fc1w, fc3w):
    n, c = feat.shape
    k = reference_index.shape[1]

    vp, kw, qw = _pre(feat, fc1w, Wq, bq, Wk, bk, Wv, bv, ww1)

    pad5 = jnp.zeros((n, 5), jnp.float32)
    tbl = jnp.concatenate(
        [vp, coord, pad5, kw, jnp.zeros((n, _TW - _OKW - _G), jnp.float32)],
        axis=1)
    tbl = jnp.pad(tbl, ((0, _NPAD - n), (0, 0)))
    ctr = jnp.concatenate([coord, pad5, qw], axis=1)

    idx_flat = reference_index.reshape(-1).astype(jnp.int32)
    idx_grp = jnp.concatenate(
        [idx_flat, jnp.zeros((_EPAD - n * k,), jnp.int32)]
    ).reshape(_EPAD // _CHUNK, _CHUNK)

    tblg = _gather(tbl, idx_grp)

    pw1p = jnp.pad(pw1, ((0, 5), (0, 0)))
    return _post(tblg, ctr, feat, pw1p, pb1, pw2, pb2, ww1, wb1, ww2,
                 wb2, fc3w)


# speedup vs baseline: 1.4986x; 1.4986x over previous
import jax
import jax.numpy as jnp
from jax import lax
from jax.experimental import pallas as pl
from jax.experimental.pallas import tpu as pltpu
from jax.experimental.pallas import tpu_sc as plsc

_N, _K, _C, _G = 10000, 16, 128, 8

_TW = 128
_OC = 64
_OKW = 72

_NW = 32
_CHUNK = 40
_EHALF = 81920
_AW = 16

_BN = 1000
_BC = 200


def _relu(x):
    return jnp.maximum(x, 0.0)


def _dot(a, b):
    return jnp.dot(a, b, preferred_element_type=jnp.float32)


def _bn_scale():
    return 1.0 / jnp.sqrt(jnp.float32(1.0) + jnp.float32(1e-5))


def _pack_bf16(v):
    bits = lax.bitcast_convert_type(v, jnp.uint32)
    c = v.shape[-1] // 2
    lo = (bits[:, 0:c] + jnp.uint32(0x8000)) >> jnp.uint32(16)
    hi = (bits[:, c:2 * c] + jnp.uint32(0x8000)) & jnp.uint32(0xFFFF0000)
    return lax.bitcast_convert_type(hi | lo, jnp.float32)


def _unpack_bf16(p):
    bits = lax.bitcast_convert_type(p, jnp.uint32)
    lo = lax.bitcast_convert_type(bits << jnp.uint32(16), jnp.float32)
    hi = lax.bitcast_convert_type(bits & jnp.uint32(0xFFFF0000), jnp.float32)
    return jnp.concatenate([lo, hi], axis=-1)


def _pre_body(feat_ref, coord_ref, fc1w_ref, wq_ref, bq_ref, wk_ref, bk_ref,
              wv_ref, bv_ref, ww1_ref, tbl_ref, ctr_ref):
    s0 = _bn_scale()
    bn = feat_ref.shape[0]
    x = _relu(s0 * _dot(feat_ref[...], fc1w_ref[...]))
    q = _relu(s0 * (_dot(x, wq_ref[...]) + bq_ref[...]))
    k = _relu(s0 * (_dot(x, wk_ref[...]) + bk_ref[...]))
    vp = _pack_bf16(_dot(x, wv_ref[...]) + bv_ref[...])
    kw = _dot(k, ww1_ref[...])
    qw = _dot(q, ww1_ref[...])
    coordb = coord_ref[...]
    pad5 = jnp.zeros((bn, 5), jnp.float32)
    tbl_ref[...] = jnp.concatenate(
        [vp, coordb, pad5, kw, jnp.zeros((bn, _TW - _OKW - _G), jnp.float32)],
        axis=1)
    ctr_ref[...] = jnp.concatenate([coordb, pad5, qw], axis=1)


def _pre(feat, coord, fc1w, Wq, bq, Wk, bk, Wv, bv, ww1):
    n, c = feat.shape
    g = ww1.shape[1]
    grid = (n // _BN,)
    full = lambda shape: pl.BlockSpec(shape, lambda i: (0, 0))
    blocked = lambda w: pl.BlockSpec((_BN, w), lambda i: (i, 0))
    return pl.pallas_call(
        _pre_body,
        grid=grid,
        in_specs=[blocked(c), blocked(3), full((c, c)), full((c, c)),
                  full((1, c)), full((c, c)), full((1, c)), full((c, c)),
                  full((1, c)), full((c, g))],
        out_specs=[blocked(_TW), blocked(_AW)],
        out_shape=[jax.ShapeDtypeStruct((n, _TW), jnp.float32),
                   jax.ShapeDtypeStruct((n, _AW), jnp.float32)],
        compiler_params=pltpu.CompilerParams(
            dimension_semantics=("parallel",)),
    )(feat, coord, fc1w, Wq, bq.reshape(1, c), Wk, bk.reshape(1, c), Wv,
      bv.reshape(1, c), ww1)


_NSLOT = 8
_LOOK = 6
_FAST_CID = 0
_CF = 64
_CS = 64


def _gather_body(tbl_hbm, idx_hbm, tblg_out, idxall, vbuf, *sems):
    gsem = sems[:_NSLOT]
    wsem = sems[_NSLOT:]
    cid = lax.axis_index("c")
    sid = lax.axis_index("s")
    is_fast = cid == _FAST_CID
    nch = lax.select(is_fast, jnp.int32(_CF), jnp.int32(_CS))
    rowbase = lax.select(is_fast, sid * _CF, 16 * _CF + sid * _CS)
    base = rowbase * _CHUNK

    @pl.when(is_fast)
    def _():
        pltpu.sync_copy(idx_hbm.at[pl.ds(rowbase, _CF)],
                        idxall.at[pl.ds(0, _CF)])

    if _CS > 0:
        @pl.when(jnp.logical_not(is_fast))
        def _():
            pltpu.sync_copy(idx_hbm.at[pl.ds(rowbase, _CS)],
                            idxall.at[pl.ds(0, _CS)])

    def start_g(slot, chunk):
        pltpu.async_copy(tbl_hbm.at[idxall.at[chunk]], vbuf.at[slot],
                         gsem[slot])

    def wait_g(slot, chunk):
        pltpu.make_async_copy(tbl_hbm.at[idxall.at[chunk]], vbuf.at[slot],
                              gsem[slot]).wait()

    def start_w(slot, chunk):
        off = base + chunk * _CHUNK
        pltpu.async_copy(vbuf.at[slot], tblg_out.at[pl.ds(off, _CHUNK)],
                         wsem[slot])

    def wait_w(slot, chunk):
        off = base + chunk * _CHUNK
        pltpu.make_async_copy(vbuf.at[slot], tblg_out.at[pl.ds(off, _CHUNK)],
                              wsem[slot]).wait()

    @pl.when(nch > 0)
    def _():
        for c in range(_LOOK):
            start_g(c % _NSLOT, c)

    def body(grp, carry):
        for j in range(_NSLOT):
            c = grp * _NSLOT + j
            wait_g(j, c)
            start_w(j, c)
            nj = (j + _LOOK) % _NSLOT

            @pl.when(c + _LOOK < nch)
            def _():
                @pl.when(c >= _NSLOT - _LOOK)
                def _():
                    wait_w(nj, c + _LOOK - _NSLOT)
                start_g(nj, c + _LOOK)
        return carry

    lax.fori_loop(0, nch // _NSLOT, body, 0)

    @pl.when(nch > 0)
    def _():
        for i in range(_LOOK):
            wait_w(_NSLOT - _LOOK + i, nch - _LOOK + i)


def _gather(tbl, idx_grp):
    mesh = plsc.VectorSubcoreMesh(core_axis_name="c", subcore_axis_name="s")
    f = pl.kernel(
        _gather_body,
        out_type=jax.ShapeDtypeStruct((_EHALF, _TW), jnp.float32),
        mesh=mesh,
        scratch_types=([pltpu.VMEM((_CF, _CHUNK), jnp.int32),
                        pltpu.VMEM((_NSLOT, _CHUNK, _TW), jnp.float32)]
                       + [pltpu.SemaphoreType.DMA] * (2 * _NSLOT)),
    )
    return f(tbl, idx_grp)


def _post_body(tblg_ref, ctr_ref, feat_ref, pw1_ref, pb1_ref,
               pw2_ref, pb2_ref, ww1_ref, wb1_ref, ww2_ref, wb2_ref,
               fc3w_ref, out_ref):
    s0 = _bn_scale()
    bc, c = out_ref.shape
    k = _K
    g = _G
    e = bc * k

    tblg = tblg_ref[...]
    auxg = tblg[:, _OC:_OC + _AW]
    ctr = ctr_ref[...]
    crep = jnp.broadcast_to(ctr[:, None, :], (bc, k, _AW)).reshape(e, _AW)

    pos8 = auxg[:, 0:8] - crep[:, 0:8]
    h = _relu(s0 * (_dot(pos8, pw1_ref[...]) + pb1_ref[...]))
    peb = _dot(h, pw2_ref[...]) + pb2_ref[...]

    kwg = auxg[:, 8:16]
    qwr = crep[:, 8:16]
    lg = kwg - qwr + _dot(peb, ww1_ref[...]) + wb1_ref[...]
    t = _relu(s0 * lg)
    wl = (_dot(t, ww2_ref[...]) + wb2_ref[...]).reshape(bc, k, g)

    m = jnp.max(wl, axis=1, keepdims=True)
    ex = jnp.exp(wl - m)
    w = (ex / jnp.sum(ex, axis=1, keepdims=True)).reshape(e, g)

    gid = lax.broadcasted_iota(jnp.int32, (g, c), 0)
    chid = lax.broadcasted_iota(jnp.int32, (g, c), 1) // (c // g)
    rexp = (gid == chid).astype(jnp.float32)
    wexp = _dot(w, rexp)

    val = _unpack_bf16(tblg[:, 0:_OC]) + peb
    attn = jnp.sum((val * wexp).reshape(bc, k, c), axis=1)
    ao = _relu(s0 * attn)
    out_ref[...] = _relu(feat_ref[...] + s0 * _dot(ao, fc3w_ref[...]))


def _post(tblg, ctr, feat, pw1p, pb1, pw2, pb2, ww1, wb1, ww2, wb2, fc3w):
    n, c = feat.shape
    g = ww1.shape[1]
    grid = (n // _BC,)
    full = lambda shape: pl.BlockSpec(shape, lambda i: (0, 0))
    return pl.pallas_call(
        _post_body,
        grid=grid,
        in_specs=[pl.BlockSpec((_BC * _K, _TW), lambda i: (i, 0)),
                  pl.BlockSpec((_BC, _AW), lambda i: (i, 0)),
                  pl.BlockSpec((_BC, c), lambda i: (i, 0)),
                  full((8, c)), full((1, c)), full((c, c)), full((1, c)),
                  full((c, g)), full((1, g)), full((g, g)), full((1, g)),
                  full((c, c))],
        out_specs=pl.BlockSpec((_BC, c), lambda i: (i, 0)),
        out_shape=jax.ShapeDtypeStruct((n, c), jnp.float32),
        compiler_params=pltpu.CompilerParams(
            dimension_semantics=("parallel",)),
    )(tblg, ctr, feat, pw1p, pb1.reshape(1, c), pw2, pb2.reshape(1, c),
      ww1, wb1.reshape(1, g), ww2, wb2.reshape(1, g), fc3w)


def kernel(feat, coord, reference_index, Wq, bq, Wk, bk, Wv, bv, pw1, pb1,
           pw2, pb2, ww1, wb1, ww2, wb2, fc1w, fc3w):
    n, c = feat.shape
    k = reference_index.shape[1]
    nh = n // 2
    eh = nh * k

    tbl, ctr = _pre(feat, coord, fc1w, Wq, bq, Wk, bk, Wv, bv, ww1)

    idx_flat = reference_index.reshape(-1).astype(jnp.int32)
    pad = jnp.zeros((_EHALF - eh,), jnp.int32)
    idx1 = jnp.concatenate([idx_flat[:eh], pad]).reshape(-1, _CHUNK)
    idx2 = jnp.concatenate([idx_flat[eh:], pad]).reshape(-1, _CHUNK)

    tblg1 = _gather(tbl, idx1)
    tblg2 = _gather(tbl, idx2)

    pw1p = jnp.pad(pw1, ((0, 5), (0, 0)))
    out1 = _post(tblg1, ctr[:nh], feat[:nh], pw1p, pb1, pw2, pb2, ww1, wb1,
                 ww2, wb2, fc3w)
    out2 = _post(tblg2, ctr[nh:], feat[nh:], pw1p, pb1, pw2, pb2, ww1, wb1,
                 ww2, wb2, fc3w)
    return jnp.concatenate([out1, out2], axis=0)
